# baseline (speedup 1.0000x reference)
import functools

import jax
import jax.numpy as jnp
from jax import lax
from jax.experimental import pallas as pl
from jax.experimental.pallas import tpu as pltpu
from jax.experimental.pallas import tpu_sc as plsc

NINP = 64

SB = 16
NBUF = 4
G = 3


def _sc_gather(idx, table):
    b_total, s = idx.shape
    v, d = table.shape
    info = plsc.get_sparse_core_info()
    nw = info.num_cores * info.num_subcores
    spw = b_total // nw
    nblk = spw // SB
    n_outer = nblk // NBUF

    mesh = plsc.VectorSubcoreMesh(core_axis_name="c", subcore_axis_name="s")

    @functools.partial(
        pl.kernel,
        mesh=mesh,
        out_type=jax.ShapeDtypeStruct((b_total, s, d), jnp.float32),
        compiler_params=pltpu.CompilerParams(use_tc_tiling_on_sc=False),
        scratch_types=[
            pltpu.VMEM((spw, s), jnp.int32),
            pltpu.VMEM((NBUF, SB, s, d), jnp.float32),
            pltpu.SemaphoreType.DMA((NBUF,)),
            pltpu.SemaphoreType.DMA((NBUF,)),
        ],
    )
    def body(idx_hbm, table_hbm, out_hbm, idx_v, rows_v, gsem, osem):
        wid = lax.axis_index("s") * info.num_cores + lax.axis_index("c")
        samp0 = wid * spw

        pltpu.sync_copy(idx_hbm.at[pl.ds(samp0, spw)], idx_v)

        def fire_block(j, b):
            for i in range(SB):
                pltpu.make_async_copy(
                    table_hbm.at[idx_v.at[j * SB + i]],
                    rows_v.at[b, i],
                    gsem.at[b],
                ).start()

        def wait_block(b):
            pltpu.make_async_copy(
                out_hbm.at[pl.ds(0, SB)], rows_v.at[b], gsem.at[b]
            ).wait()

        for k in range(G):
            fire_block(k, k)

        def outer(g0, carry):
            for b in range(NBUF):
                j = g0 * NBUF + b
                bn = (b + G) % NBUF
                wait_block(b)
                pltpu.make_async_copy(
                    rows_v.at[b],
                    out_hbm.at[pl.ds(samp0 + j * SB, SB)],
                    osem.at[b],
                ).start()
                @pl.when(jnp.logical_and(j + G < nblk, j + G >= NBUF))
                def _():
                    pltpu.make_async_copy(
                        rows_v.at[bn],
                        out_hbm.at[pl.ds(samp0 + (j + G - NBUF) * SB, SB)],
                        osem.at[bn],
                    ).wait()

                @pl.when(j + G < nblk)
                def _():
                    fire_block(j + G, bn)
            return carry

        lax.fori_loop(0, n_outer, outer, 0)

        for b in range(NBUF):
            j = nblk - NBUF + b
            pltpu.make_async_copy(
                rows_v.at[b],
                out_hbm.at[pl.ds(samp0 + j * SB, SB)],
                osem.at[b],
            ).wait()

    return body(idx, table)


def kernel(input, encoder_weight):
    return _sc_gather(input.astype(jnp.int32), encoder_weight)


# speedup vs baseline: 1.3800x; 1.3800x over previous
import functools

import jax
import jax.numpy as jnp
from jax import lax
from jax.experimental import layout as jax_layout
from jax.experimental import pallas as pl
from jax.experimental.pallas import tpu as pltpu
from jax.experimental.pallas import tpu_sc as plsc

NINP = 64

SB = 16
NBUF = 4
G = 3


def _sc_gather(idx, table):
    b_total, s = idx.shape
    v, d = table.shape
    info = plsc.get_sparse_core_info()
    nw = info.num_cores * info.num_subcores
    spw = b_total // nw
    nblk = spw // SB
    n_outer = nblk // NBUF

    mesh = plsc.VectorSubcoreMesh(core_axis_name="c", subcore_axis_name="s")

    @functools.partial(
        pl.kernel,
        mesh=mesh,
        out_type=jax.ShapeDtypeStruct((b_total, s, d), jnp.float32),
        compiler_params=pltpu.CompilerParams(use_tc_tiling_on_sc=False),
        scratch_types=[
            pltpu.VMEM((spw, s), jnp.int32),
            pltpu.VMEM((NBUF, SB, s, d), jnp.float32),
            pltpu.SemaphoreType.DMA((NBUF,)),
            pltpu.SemaphoreType.DMA((NBUF,)),
        ],
    )
    def body(idx_hbm, table_hbm, out_hbm, idx_v, rows_v, gsem, osem):
        wid = lax.axis_index("s") * info.num_cores + lax.axis_index("c")
        samp0 = wid * spw

        pltpu.sync_copy(idx_hbm.at[pl.ds(samp0, spw)], idx_v)

        def fire_block(j, b):
            for i in range(SB):
                pltpu.make_async_copy(
                    table_hbm.at[idx_v.at[j * SB + i]],
                    rows_v.at[b, i],
                    gsem.at[b],
                ).start()

        def wait_block(b):
            pltpu.make_async_copy(
                out_hbm.at[pl.ds(0, SB)], rows_v.at[b], gsem.at[b]
            ).wait()

        for k in range(G):
            fire_block(k, k)

        def outer(g0, carry):
            for b in range(NBUF):
                j = g0 * NBUF + b
                bn = (b + G) % NBUF
                wait_block(b)
                pltpu.make_async_copy(
                    rows_v.at[b],
                    out_hbm.at[pl.ds(samp0 + j * SB, SB)],
                    osem.at[b],
                ).start()
                @pl.when(jnp.logical_and(j + G < nblk, j + G >= NBUF))
                def _():
                    pltpu.make_async_copy(
                        rows_v.at[bn],
                        out_hbm.at[pl.ds(samp0 + (j + G - NBUF) * SB, SB)],
                        osem.at[bn],
                    ).wait()

                @pl.when(j + G < nblk)
                def _():
                    fire_block(j + G, bn)
            return carry

        lax.fori_loop(0, n_outer, outer, 0)

        for b in range(NBUF):
            j = nblk - NBUF + b
            pltpu.make_async_copy(
                rows_v.at[b],
                out_hbm.at[pl.ds(samp0 + j * SB, SB)],
                osem.at[b],
            ).wait()

    return body(idx, table)


def kernel(input, encoder_weight):
    table = jax_layout.with_layout_constraint(
        encoder_weight,
        jax_layout.Layout(major_to_minor=(0, 1)),
    )
    return _sc_gather(input.astype(jnp.int32) * 2, table)


# speedup vs baseline: 1.3827x; 1.0019x over previous
import functools

import jax
import jax.numpy as jnp
from jax import lax
from jax.experimental import layout as jax_layout
from jax.experimental import pallas as pl
from jax.experimental.pallas import tpu as pltpu
from jax.experimental.pallas import tpu_sc as plsc

NINP = 64

SB = 16
NBUF = 4
G = 3


def _sc_gather(idx, table):
    b_total, s = idx.shape
    v, d = table.shape
    info = plsc.get_sparse_core_info()
    nw = info.num_cores * info.num_subcores
    spw = b_total // nw
    nblk = spw // SB
    n_outer = nblk // NBUF

    mesh = plsc.VectorSubcoreMesh(core_axis_name="c", subcore_axis_name="s")

    @functools.partial(
        pl.kernel,
        mesh=mesh,
        out_type=jax.ShapeDtypeStruct((b_total, s, d), jnp.float32),
        compiler_params=pltpu.CompilerParams(use_tc_tiling_on_sc=False),
        scratch_types=[
            pltpu.VMEM((spw, s), jnp.int32),
            pltpu.VMEM((NBUF, SB, s, d), jnp.float32),
            pltpu.SemaphoreType.DMA((NBUF,)),
            pltpu.SemaphoreType.DMA((NBUF,)),
        ],
    )
    def body(idx_hbm, table_hbm, out_hbm, idx_v, rows_v, gsem, osem):
        wid = lax.axis_index("s") * info.num_cores + lax.axis_index("c")
        samp0 = wid * spw

        pltpu.sync_copy(idx_hbm.at[pl.ds(samp0, spw)], idx_v)

        def fire_block(j, b):
            for i in range(SB):
                pltpu.make_async_copy(
                    table_hbm.at[idx_v.at[j * SB + i]],
                    rows_v.at[b, i],
                    gsem.at[b],
                ).start()

        def wait_block(b):
            pltpu.make_async_copy(
                out_hbm.at[pl.ds(0, SB)], rows_v.at[b], gsem.at[b]
            ).wait()

        for k in range(G):
            fire_block(k, k)

        def outer(g0, carry):
            for b in range(NBUF):
                j = g0 * NBUF + b
                bn = (b + G) % NBUF
                wait_block(b)
                pltpu.make_async_copy(
                    rows_v.at[b],
                    out_hbm.at[pl.ds(samp0 + j * SB, SB)],
                    osem.at[b],
                ).start()
                @pl.when(jnp.logical_and(j + G < nblk, j + G >= NBUF))
                def _():
                    pltpu.make_async_copy(
                        rows_v.at[bn],
                        out_hbm.at[pl.ds(samp0 + (j + G - NBUF) * SB, SB)],
                        osem.at[bn],
                    ).wait()

                @pl.when(j + G < nblk)
                def _():
                    fire_block(j + G, bn)
            return carry

        lax.fori_loop(0, n_outer, outer, 0)

        for b in range(NBUF):
            j = nblk - NBUF + b
            pltpu.make_async_copy(
                rows_v.at[b],
                out_hbm.at[pl.ds(samp0 + j * SB, SB)],
                osem.at[b],
            ).wait()

    return body(idx, table)


def kernel(input, encoder_weight):
    table = jax_layout.with_layout_constraint(
        encoder_weight,
        jax_layout.Layout(major_to_minor=(0, 1), tiling=((8,),)),
    )
    return _sc_gather(input.astype(jnp.int32) * 2, table)


# speedup vs baseline: 1.7638x; 1.2757x over previous
import functools

import jax
import jax.numpy as jnp
from jax import lax
from jax.experimental import layout as jax_layout
from jax.experimental import pallas as pl
from jax.experimental.pallas import tpu as pltpu
from jax.experimental.pallas import tpu_sc as plsc

NINP = 64

SB = 16
NBUF = 4
G = 3


def _sc_gather(idx, table):
    b_total, s = idx.shape
    v, d = table.shape
    info = plsc.get_sparse_core_info()
    nw = info.num_cores * info.num_subcores
    spw = b_total // nw
    nblk = spw // SB
    n_outer = nblk // NBUF

    mesh = plsc.VectorSubcoreMesh(core_axis_name="c", subcore_axis_name="s")
    sp = (s + 7) // 8 * 8
    dp = 128

    @functools.partial(
        pl.kernel,
        mesh=mesh,
        out_type=jax.ShapeDtypeStruct((b_total, sp, dp), jnp.float32),
        compiler_params=pltpu.CompilerParams(use_tc_tiling_on_sc=False),
        scratch_types=[
            pltpu.VMEM((spw, s), jnp.int32),
            pltpu.VMEM((NBUF, SB, s, d), jnp.float32),
            pltpu.SemaphoreType.DMA((NBUF,)),
            pltpu.SemaphoreType.DMA((NBUF,)),
        ],
    )
    def body(idx_hbm, table_hbm, out_hbm, idx_v, rows_v, gsem, osem):
        wid = lax.axis_index("s") * info.num_cores + lax.axis_index("c")
        samp0 = wid * spw

        pltpu.sync_copy(idx_hbm.at[pl.ds(samp0, spw)], idx_v)

        def fire_block(j, b):
            for i in range(SB):
                pltpu.make_async_copy(
                    table_hbm.at[idx_v.at[j * SB + i]],
                    rows_v.at[b, i],
                    gsem.at[b],
                ).start()

        def wait_block(b):
            pltpu.make_async_copy(
                out_hbm.at[pl.ds(0, SB), pl.ds(0, s), pl.ds(0, d)],
                rows_v.at[b], gsem.at[b],
            ).wait()

        for k in range(G):
            fire_block(k, k)

        def outer(g0, carry):
            for b in range(NBUF):
                j = g0 * NBUF + b
                bn = (b + G) % NBUF
                wait_block(b)
                pltpu.make_async_copy(
                    rows_v.at[b],
                    out_hbm.at[pl.ds(samp0 + j * SB, SB), pl.ds(0, s),
                               pl.ds(0, d)],
                    osem.at[b],
                ).start()
                @pl.when(jnp.logical_and(j + G < nblk, j + G >= NBUF))
                def _():
                    pltpu.make_async_copy(
                        rows_v.at[bn],
                        out_hbm.at[pl.ds(samp0 + (j + G - NBUF) * SB, SB),
                                   pl.ds(0, s), pl.ds(0, d)],
                        osem.at[bn],
                    ).wait()

                @pl.when(j + G < nblk)
                def _():
                    fire_block(j + G, bn)
            return carry

        lax.fori_loop(0, n_outer, outer, 0)

        for b in range(NBUF):
            j = nblk - NBUF + b
            pltpu.make_async_copy(
                rows_v.at[b],
                out_hbm.at[pl.ds(samp0 + j * SB, SB), pl.ds(0, s),
                           pl.ds(0, d)],
                osem.at[b],
            ).wait()

    out_pad = body(idx, table)
    return lax.slice(out_pad, (0, 0, 0), (b_total, s, d))


def kernel(input, encoder_weight):
    table = jax_layout.with_layout_constraint(
        encoder_weight,
        jax_layout.Layout(major_to_minor=(0, 1), tiling=((8,),)),
    )
    return _sc_gather(input.astype(jnp.int32) * 2, table)
